# baseline (speedup 1.0000x reference)
import jax
import jax.numpy as jnp
from jax import lax
from jax.experimental import pallas as pl
from jax.experimental.pallas import tpu as pltpu
from jax.experimental.pallas import tpu_sc as plsc

N = 10000
E = 320000
D = 128
NCLS = 8

NPAD = 10112
EPAD = 327680
ROWS_PER_TILE = NPAD // 16
_ZS = [64] * 9 + [56]


def _stages(k, slab):
    if k <= slab:
        return [(k, 0)]
    assert k % slab == 0
    return [(slab, i * slab) for i in range(k // slab)]


def _zero_buf(ref, nrows, width, lanes, dtype):
    z = jnp.zeros((lanes,), dtype)

    def row(i, _):
        for j in range(width // lanes):
            ref[i, pl.ds(j * lanes, lanes)] = z
        return ()

    lax.fori_loop(0, nrows, row, ())


def _make_segsum(dtype, width, ch, nbuf, k0, slab, with_counts):
    assert 16 * k0 * ch == EPAD
    dep = nbuf // 2
    assert (k0 <= slab or k0 % slab == 0) and k0 % nbuf == 0 and k0 >= nbuf
    assert slab % nbuf == 0
    lanes = 32 if dtype == jnp.bfloat16 else 16
    mesh = plsc.VectorSubcoreMesh(core_axis_name="c", subcore_axis_name="s")

    zsizes = [ch] * (ROWS_PER_TILE // ch)
    if ROWS_PER_TILE % ch:
        zsizes.append(ROWS_PER_TILE % ch)

    def body(table, srcm, dstm, *rest):
        rest = list(rest)
        out_x = rest.pop(0)
        out_c = rest.pop(0) if with_counts else None
        isrc = rest.pop(0)
        idst = rest.pop(0)
        bufs = [rest.pop(0) for _ in range(nbuf)]
        gsems = [rest.pop(0) for _ in range(nbuf)]
        ssems = [rest.pop(0) for _ in range(nbuf)]
        if with_counts:
            ones, zb16, osem = rest.pop(0), rest.pop(0), rest.pop(0)
        acc = rest.pop(0)
        acc_c = rest.pop(0) if with_counts else None
        c = lax.axis_index("c")
        s = lax.axis_index("s")

        def prologue():
            _zero_buf(bufs[0], ch, width, lanes, dtype)
            if with_counts:
                _zero_buf(zb16, ch, 16, 16, jnp.float32)
                lane = lax.iota(jnp.int32, 16)
                one_row = jnp.where(lane == 0, 1.0, 0.0).astype(jnp.float32)

                def ones_row(i, _):
                    ones[i, pl.ds(0, 16)] = one_row
                    return ()

                lax.fori_loop(0, ch, ones_row, ())

            pending = []
            rel = 0
            for i, sz in enumerate(zsizes):
                sem = gsems[i % nbuf]
                src = bufs[0].at[pl.ds(0, sz)]
                dst = acc.at[pl.ds(s * ROWS_PER_TILE + rel, sz)]
                pltpu.async_copy(src, dst, sem)
                pending.append((src, dst, sem))
                if with_counts:
                    sem2 = ssems[i % nbuf]
                    src2 = zb16.at[pl.ds(0, sz)]
                    dst2 = acc_c.at[pl.ds(s * ROWS_PER_TILE + rel, sz)]
                    pltpu.async_copy(src2, dst2, sem2)
                    pending.append((src2, dst2, sem2))
                rel += sz
            for src, dst, sem in pending:
                pltpu.make_async_copy(src, dst, sem).wait()

        def fire_g(r, k):
            pltpu.async_copy(table.at[isrc.at[k]], bufs[r], gsems[r])

        def wait_g(r):
            pltpu.make_async_copy(table.at[pl.ds(0, ch)], bufs[r], gsems[r]).wait()

        def fire_s(r, k):
            pltpu.async_copy(bufs[r], acc.at[idst.at[k]], ssems[r], add=True)

        def wait_s(r):
            pltpu.make_async_copy(bufs[r], acc.at[pl.ds(0, ch)], ssems[r]).wait()

        def run_range(base_chunk, nch):
            pltpu.sync_copy(srcm.at[pl.ds(base_chunk, nch)], isrc.at[pl.ds(0, nch)])
            pltpu.sync_copy(dstm.at[pl.ds(base_chunk, nch)], idst.at[pl.ds(0, nch)])
            for r in range(dep):
                fire_g(r, r)

            def group(j, _):
                kb = j * nbuf
                for r in range(nbuf):
                    k = kb + r
                    wait_g(r)
                    fire_s(r, k)
                    if with_counts:
                        pltpu.async_copy(ones, acc_c.at[idst.at[k]], osem, add=True)
                    r2 = (r + dep) % nbuf

                    @pl.when((k >= dep) & (k + dep < nch))
                    def _(r2=r2):
                        wait_s(r2)

                    @pl.when(k + dep < nch)
                    def _(r2=r2, k=k):
                        fire_g(r2, k + dep)
                return ()

            lax.fori_loop(0, nch // nbuf, group, ())
            for r in range(nbuf):
                wait_s(r)
            if with_counts:
                def drain(i, _):
                    pltpu.make_async_copy(ones, acc_c.at[pl.ds(0, ch)], osem).wait()
                    return ()

                lax.fori_loop(0, nch, drain, ())

        @pl.when(c == 0)
        def _():
            prologue()
            plsc.subcore_barrier()
            for sz, rel in _stages(k0, slab):
                run_range(s * k0 + rel, sz)
            plsc.subcore_barrier()
            pltpu.sync_copy(acc.at[pl.ds(s * ROWS_PER_TILE, ROWS_PER_TILE)],
                            out_x.at[pl.ds(s * ROWS_PER_TILE, ROWS_PER_TILE)])
            if with_counts:
                pltpu.sync_copy(acc_c.at[pl.ds(s * ROWS_PER_TILE, ROWS_PER_TILE)],
                                out_c.at[pl.ds(s * ROWS_PER_TILE, ROWS_PER_TILE)])

    out_type = [jax.ShapeDtypeStruct((NPAD, width), dtype)]
    scratch = [pltpu.VMEM((slab, ch), jnp.int32), pltpu.VMEM((slab, ch), jnp.int32)]
    scratch += [pltpu.VMEM((ch, width), dtype)] * nbuf
    scratch += [pltpu.SemaphoreType.DMA] * (2 * nbuf)
    if with_counts:
        out_type.append(jax.ShapeDtypeStruct((NPAD, 16), jnp.float32))
        scratch += [pltpu.VMEM((ch, 16), jnp.float32),
                    pltpu.VMEM((ch, 16), jnp.float32),
                    pltpu.SemaphoreType.DMA]
    scratch.append(pltpu.VMEM_SHARED((NPAD, width), dtype))
    if with_counts:
        scratch.append(pltpu.VMEM_SHARED((NPAD, 16), jnp.float32))

    return pl.kernel(
        body,
        out_type=out_type if with_counts else out_type[0],
        mesh=mesh,
        scratch_types=scratch,
        compiler_params=pltpu.CompilerParams(use_tc_tiling_on_sc=False),
    )


_CH1 = 64
_CH2 = 128
_segsum1 = _make_segsum(jnp.bfloat16, D, _CH1, 8, 320, 160, True)
_segsum2 = _make_segsum(jnp.float32, 16, _CH2, 8, 160, 160, False)

_RB = ROWS_PER_TILE
_R = 1000


def _layer1_body(px_ref, pc_ref, x_ref, w1lt_ref, w1rt_ref, b1l_ref, m16t_ref,
                 e8_ref, h_ref, g_ref):
    cnt = jnp.maximum(pc_ref[:, 0:1], 1.0)
    mean = px_ref[...].astype(jnp.float32) / cnt
    h = mean @ w1lt_ref[...] + x_ref[...] @ w1rt_ref[...] + b1l_ref[...]
    h = jnp.maximum(h, 0.0)
    h_ref[...] = h
    rid = pl.program_id(0) * _RB + lax.broadcasted_iota(jnp.int32, (_RB, 1), 0)
    g_ref[...] = jnp.where(rid < N, h @ m16t_ref[...] + e8_ref[...], 0.0)


def _head_body(p2_ref, h_ref, flat_ref, wht_ref, wfot_ref, btot_ref, out_ref):
    p = p2_ref[...]
    cnt = jnp.maximum(p[:, NCLS:NCLS + 1], 1.0)
    seg = p[:, :NCLS] / cnt
    out_ref[...] = seg + h_ref[...] @ wht_ref[...] + flat_ref[...] @ wfot_ref[...] + btot_ref[...]


def _full(shape):
    return pl.BlockSpec(shape, lambda i: tuple(0 for _ in shape))


_layer1 = pl.pallas_call(
    _layer1_body,
    grid=(NPAD // _RB,),
    in_specs=[
        pl.BlockSpec((_RB, D), lambda i: (i, 0)),
        pl.BlockSpec((_RB, 16), lambda i: (i, 0)),
        pl.BlockSpec((_RB, D), lambda i: (i, 0)),
        _full((D, D)),
        _full((D, D)),
        _full((1, D)),
        _full((D, 16)),
        _full((1, 16)),
    ],
    out_specs=[
        pl.BlockSpec((_RB, D), lambda i: (i, 0)),
        pl.BlockSpec((_RB, 16), lambda i: (i, 0)),
    ],
    out_shape=[
        jax.ShapeDtypeStruct((NPAD, D), jnp.float32),
        jax.ShapeDtypeStruct((NPAD, 16), jnp.float32),
    ],
)

_head = pl.pallas_call(
    _head_body,
    grid=(N // _R,),
    in_specs=[
        pl.BlockSpec((_R, 16), lambda i: (i, 0)),
        pl.BlockSpec((_R, D), lambda i: (i, 0)),
        pl.BlockSpec((_R, 32), lambda i: (i, 0)),
        _full((D, NCLS)),
        _full((32, NCLS)),
        _full((1, NCLS)),
    ],
    out_specs=pl.BlockSpec((_R, NCLS), lambda i: (i, 0)),
    out_shape=jax.ShapeDtypeStruct((N, NCLS), jnp.float32),
)


def _pad_edges(edge_index, ch):
    src = jnp.concatenate([edge_index[0], jnp.zeros((EPAD - E,), jnp.int32)])
    dst = jnp.concatenate([edge_index[1], jnp.full((EPAD - E,), N, jnp.int32)])
    return src.reshape(EPAD // ch, ch), dst.reshape(EPAD // ch, ch)


@jax.jit
def kernel(x, flat, edge_index1, edge_index2, W1l, b1l, W1r, W2l, b2l, W2r, Wf, bf, Wo, bo):
    src1, dst1 = _pad_edges(edge_index1, _CH1)
    src2, dst2 = _pad_edges(edge_index2, _CH2)

    WoA = Wo[:, :D]
    WoB = Wo[:, D:]
    M = WoA @ W2l
    m16t = jnp.concatenate([M, jnp.zeros((8, D), jnp.float32)]).T
    e8 = jnp.zeros((1, 16), jnp.float32).at[0, NCLS].set(1.0)
    wht = (WoA @ W2r).T
    wfot = (WoB @ Wf).T
    btot = (bo + WoA @ b2l + WoB @ bf).reshape(1, NCLS)

    px, pc = _segsum1(x.astype(jnp.bfloat16), src1, dst1)
    h, gtab = _layer1(px, pc, x, W1l.T, W1r.T, b1l.reshape(1, D), m16t, e8)
    p2 = _segsum2(gtab, src2, dst2)
    return _head(p2, h, flat, wht, wfot, btot)


# speedup vs baseline: 1.1661x; 1.1661x over previous
import jax
import jax.numpy as jnp
from jax import lax
from jax.experimental import pallas as pl
from jax.experimental.pallas import tpu as pltpu
from jax.experimental.pallas import tpu_sc as plsc

N = 10000
E = 320000
D = 128
NCLS = 8

NPAD = 10112
EPAD = 327680
ROWS_PER_TILE = NPAD // 16
_ZS = [64] * 9 + [56]


def _stages(k, slab):
    if k <= slab:
        return [(k, 0)]
    assert k % slab == 0
    return [(slab, i * slab) for i in range(k // slab)]


def _zero_buf(ref, nrows, width, lanes, dtype):
    z = jnp.zeros((lanes,), dtype)

    def row(i, _):
        for j in range(width // lanes):
            ref[i, pl.ds(j * lanes, lanes)] = z
        return ()

    lax.fori_loop(0, nrows, row, ())


def _make_segsum(dtype, width, ch, nbuf, k0, k1, slab, with_counts):
    assert 16 * (k0 + k1) * ch == EPAD
    dep = nbuf // 2
    for k in (k0, k1):
        assert (k <= slab or k % slab == 0) and k % nbuf == 0 and k >= nbuf
    assert slab % nbuf == 0
    lanes = 32 if dtype == jnp.bfloat16 else 16
    mesh = plsc.VectorSubcoreMesh(core_axis_name="c", subcore_axis_name="s")

    zsizes = [ch] * (ROWS_PER_TILE // ch)
    if ROWS_PER_TILE % ch:
        zsizes.append(ROWS_PER_TILE % ch)

    def body(table, srcm, dstm, *rest):
        rest = list(rest)
        out_x = rest.pop(0)
        out_c = rest.pop(0) if with_counts else None
        isrc = rest.pop(0)
        idst = rest.pop(0)
        bufs = [rest.pop(0) for _ in range(nbuf)]
        gsems = [rest.pop(0) for _ in range(nbuf)]
        ssems = [rest.pop(0) for _ in range(nbuf)]
        if with_counts:
            ones, zb16, osem = rest.pop(0), rest.pop(0), rest.pop(0)
        acc = rest.pop(0)
        acc_c = rest.pop(0) if with_counts else None
        c = lax.axis_index("c")
        s = lax.axis_index("s")

        _zero_buf(bufs[0], ch, width, lanes, dtype)
        if with_counts:
            _zero_buf(zb16, ch, 16, 16, jnp.float32)
            lane = lax.iota(jnp.int32, 16)
            one_row = jnp.where(lane == 0, 1.0, 0.0).astype(jnp.float32)

            def ones_row(i, _):
                ones[i, pl.ds(0, 16)] = one_row
                return ()

            lax.fori_loop(0, ch, ones_row, ())

        rel = 0
        for sz in zsizes:
            pltpu.sync_copy(bufs[0].at[pl.ds(0, sz)],
                            acc.at[pl.ds(s * ROWS_PER_TILE + rel, sz)])
            if with_counts:
                pltpu.sync_copy(zb16.at[pl.ds(0, sz)],
                                acc_c.at[pl.ds(s * ROWS_PER_TILE + rel, sz)])
            rel += sz
        plsc.subcore_barrier()

        def fire_g(r, k):
            pltpu.async_copy(table.at[isrc.at[k]], bufs[r], gsems[r])

        def wait_g(r):
            pltpu.make_async_copy(table.at[pl.ds(0, ch)], bufs[r], gsems[r]).wait()

        def fire_s(r, k):
            pltpu.async_copy(bufs[r], acc.at[idst.at[k]], ssems[r], add=True)

        def wait_s(r):
            pltpu.make_async_copy(bufs[r], acc.at[pl.ds(0, ch)], ssems[r]).wait()

        def run_range(base_chunk, nch):
            pltpu.sync_copy(srcm.at[pl.ds(base_chunk, nch)], isrc.at[pl.ds(0, nch)])
            pltpu.sync_copy(dstm.at[pl.ds(base_chunk, nch)], idst.at[pl.ds(0, nch)])
            for r in range(dep):
                fire_g(r, r)

            def group(j, _):
                kb = j * nbuf
                for r in range(nbuf):
                    k = kb + r
                    wait_g(r)
                    fire_s(r, k)
                    if with_counts:
                        pltpu.async_copy(ones, acc_c.at[idst.at[k]], osem, add=True)
                    r2 = (r + dep) % nbuf

                    @pl.when((k >= dep) & (k + dep < nch))
                    def _(r2=r2):
                        wait_s(r2)

                    @pl.when(k + dep < nch)
                    def _(r2=r2, k=k):
                        fire_g(r2, k + dep)
                return ()

            lax.fori_loop(0, nch // nbuf, group, ())
            for r in range(nbuf):
                wait_s(r)
            if with_counts:
                def drain(i, _):
                    pltpu.make_async_copy(ones, acc_c.at[pl.ds(0, ch)], osem).wait()
                    return ()

                lax.fori_loop(0, nch, drain, ())

        @pl.when(c == 0)
        def _():
            for sz, rel in _stages(k0, slab):
                run_range(s * k0 + rel, sz)

        @pl.when(c == 1)
        def _():
            for sz, rel in _stages(k1, slab):
                run_range(16 * k0 + s * k1 + rel, sz)

        plsc.subcore_barrier()
        pltpu.sync_copy(acc.at[pl.ds(s * ROWS_PER_TILE, ROWS_PER_TILE)],
                        out_x.at[c, pl.ds(s * ROWS_PER_TILE, ROWS_PER_TILE)])
        if with_counts:
            pltpu.sync_copy(acc_c.at[pl.ds(s * ROWS_PER_TILE, ROWS_PER_TILE)],
                            out_c.at[c, pl.ds(s * ROWS_PER_TILE, ROWS_PER_TILE)])

    out_type = [jax.ShapeDtypeStruct((2, NPAD, width), dtype)]
    scratch = [pltpu.VMEM((slab, ch), jnp.int32), pltpu.VMEM((slab, ch), jnp.int32)]
    scratch += [pltpu.VMEM((ch, width), dtype)] * nbuf
    scratch += [pltpu.SemaphoreType.DMA] * (2 * nbuf)
    if with_counts:
        out_type.append(jax.ShapeDtypeStruct((2, NPAD, 16), jnp.float32))
        scratch += [pltpu.VMEM((ch, 16), jnp.float32),
                    pltpu.VMEM((ch, 16), jnp.float32),
                    pltpu.SemaphoreType.DMA]
    scratch.append(pltpu.VMEM_SHARED((NPAD, width), dtype))
    if with_counts:
        scratch.append(pltpu.VMEM_SHARED((NPAD, 16), jnp.float32))

    return pl.kernel(
        body,
        out_type=out_type if with_counts else out_type[0],
        mesh=mesh,
        scratch_types=scratch,
        compiler_params=pltpu.CompilerParams(use_tc_tiling_on_sc=False),
    )


_CH1 = 64
_CH2 = 128
_segsum1 = _make_segsum(jnp.bfloat16, D, _CH1, 8, 272, 48, 136, True)
_segsum2 = _make_segsum(jnp.float32, 16, _CH2, 8, 120, 40, 120, False)

_RB = ROWS_PER_TILE
_R = 1000


def _layer1_body(px_ref, pc_ref, x_ref, w1lt_ref, w1rt_ref, b1l_ref, m16t_ref,
                 e8_ref, h_ref, g_ref):
    cnt = jnp.maximum(pc_ref[0, :, 0:1] + pc_ref[1, :, 0:1], 1.0)
    mean = (px_ref[0].astype(jnp.float32) + px_ref[1].astype(jnp.float32)) / cnt
    h = mean @ w1lt_ref[...] + x_ref[...] @ w1rt_ref[...] + b1l_ref[...]
    h = jnp.maximum(h, 0.0)
    h_ref[...] = h
    rid = pl.program_id(0) * _RB + lax.broadcasted_iota(jnp.int32, (_RB, 1), 0)
    g_ref[...] = jnp.where(rid < N, h @ m16t_ref[...] + e8_ref[...], 0.0)


def _head_body(p2_ref, h_ref, flat_ref, wht_ref, wfot_ref, btot_ref, out_ref):
    p = p2_ref[0] + p2_ref[1]
    cnt = jnp.maximum(p[:, NCLS:NCLS + 1], 1.0)
    seg = p[:, :NCLS] / cnt
    out_ref[...] = seg + h_ref[...] @ wht_ref[...] + flat_ref[...] @ wfot_ref[...] + btot_ref[...]


def _full(shape):
    return pl.BlockSpec(shape, lambda i: tuple(0 for _ in shape))


_layer1 = pl.pallas_call(
    _layer1_body,
    grid=(NPAD // _RB,),
    in_specs=[
        pl.BlockSpec((2, _RB, D), lambda i: (0, i, 0)),
        pl.BlockSpec((2, _RB, 16), lambda i: (0, i, 0)),
        pl.BlockSpec((_RB, D), lambda i: (i, 0)),
        _full((D, D)),
        _full((D, D)),
        _full((1, D)),
        _full((D, 16)),
        _full((1, 16)),
    ],
    out_specs=[
        pl.BlockSpec((_RB, D), lambda i: (i, 0)),
        pl.BlockSpec((_RB, 16), lambda i: (i, 0)),
    ],
    out_shape=[
        jax.ShapeDtypeStruct((NPAD, D), jnp.float32),
        jax.ShapeDtypeStruct((NPAD, 16), jnp.float32),
    ],
)

_head = pl.pallas_call(
    _head_body,
    grid=(N // _R,),
    in_specs=[
        pl.BlockSpec((2, _R, 16), lambda i: (0, i, 0)),
        pl.BlockSpec((_R, D), lambda i: (i, 0)),
        pl.BlockSpec((_R, 32), lambda i: (i, 0)),
        _full((D, NCLS)),
        _full((32, NCLS)),
        _full((1, NCLS)),
    ],
    out_specs=pl.BlockSpec((_R, NCLS), lambda i: (i, 0)),
    out_shape=jax.ShapeDtypeStruct((N, NCLS), jnp.float32),
)


def _pad_edges(edge_index, ch):
    src = jnp.concatenate([edge_index[0], jnp.zeros((EPAD - E,), jnp.int32)])
    dst = jnp.concatenate([edge_index[1], jnp.full((EPAD - E,), N, jnp.int32)])
    return src.reshape(EPAD // ch, ch), dst.reshape(EPAD // ch, ch)


@jax.jit
def kernel(x, flat, edge_index1, edge_index2, W1l, b1l, W1r, W2l, b2l, W2r, Wf, bf, Wo, bo):
    src1, dst1 = _pad_edges(edge_index1, _CH1)
    src2, dst2 = _pad_edges(edge_index2, _CH2)

    WoA = Wo[:, :D]
    WoB = Wo[:, D:]
    M = WoA @ W2l
    m16t = jnp.concatenate([M, jnp.zeros((8, D), jnp.float32)]).T
    e8 = jnp.zeros((1, 16), jnp.float32).at[0, NCLS].set(1.0)
    wht = (WoA @ W2r).T
    wfot = (WoB @ Wf).T
    btot = (bo + WoA @ b2l + WoB @ bf).reshape(1, NCLS)

    px, pc = _segsum1(x.astype(jnp.bfloat16), src1, dst1)
    h, gtab = _layer1(px, pc, x, W1l.T, W1r.T, b1l.reshape(1, D), m16t, e8)
    p2 = _segsum2(gtab, src2, dst2)
    return _head(p2, h, flat, wht, wfot, btot)
